# baseline (speedup 1.0000x reference)
import functools

import jax
import jax.numpy as jnp
from jax import lax
from jax.experimental import pallas as pl
from jax.experimental.pallas import tpu as pltpu
from jax.experimental.pallas import tpu_sc as plsc

_BETA_THRESHOLD = 0.3
_DISTANCE_THRESHOLD = 0.5
_N = 20000
_ROWS = 8
_COLS = 2560
_P = _ROWS * _COLS

_NW = 32
_BW = _P // _NW
_NCH = _BW // 128


def _greedy_body(cx_ref, cy_ref, cz_ref, t2_ref, b_ref,
                 assign_ref, alpha_ref, mb_ref):
    shape = (_ROWS, _COLS)
    neg1 = jnp.full(shape, -1, jnp.int32)
    assign_ref[...] = neg1
    alpha_ref[...] = neg1
    mb_ref[...] = b_ref[...]

    row = lax.broadcasted_iota(jnp.int32, shape, 0)
    col = lax.broadcasted_iota(jnp.int32, shape, 1)
    idx = row * _COLS + col

    cx = cx_ref[...]
    cy = cy_ref[...]
    cz = cz_ref[...]
    t2 = t2_ref[...]

    def cond_fn(carry):
        _, m = carry
        return m > _BETA_THRESHOLD

    def body_fn(carry):
        k, m = carry
        mb = mb_ref[...]
        a = jnp.min(jnp.where(mb >= m, idx, _P))
        sel = idx == a
        ninf = jnp.float32(-jnp.inf)
        cxa = jnp.max(jnp.where(sel, cx, ninf))
        cya = jnp.max(jnp.where(sel, cy, ninf))
        cza = jnp.max(jnp.where(sel, cz, ninf))
        t2a = jnp.max(jnp.where(sel, t2, ninf))
        dx = cx - cxa
        dy = cy - cya
        dz = cz - cza
        d2 = (dx * dx + dy * dy) + dz * dz
        within = (d2 <= t2a) & (assign_ref[...] < 0)
        assign_ref[...] = jnp.where(within, k, assign_ref[...])
        alpha_ref[...] = jnp.where(within, a, alpha_ref[...])
        mb2 = jnp.where(within, jnp.float32(-2.0), mb)
        mb_ref[...] = mb2
        return k + jnp.int32(1), jnp.max(mb2)

    m0 = jnp.max(b_ref[...])
    lax.while_loop(cond_fn, body_fn, (jnp.int32(0), m0))


def _sc_gather_body(aidx_hbm, cx_hbm, cy_hbm, cz_hbm,
                    ox_hbm, oy_hbm, oz_hbm,
                    aidx_v, idx_v, gx_v, gy_v, gz_v, sem):
    wid = lax.axis_index("s") * 2 + lax.axis_index("c")
    pltpu.sync_copy(aidx_hbm.at[wid], aidx_v)
    for j in range(_BW // 16):
        r, c = j // 8, (j % 8) * 16
        a = aidx_v[r, pl.ds(c, 16)]
        idx_v[r, pl.ds(c, 16)] = jnp.maximum(a, 0)
    for r in range(_NCH):
        pltpu.async_copy(cx_hbm.at[idx_v.at[r]], gx_v.at[r], sem).wait()
        pltpu.async_copy(cy_hbm.at[idx_v.at[r]], gy_v.at[r], sem).wait()
        pltpu.async_copy(cz_hbm.at[idx_v.at[r]], gz_v.at[r], sem).wait()
    zero = jnp.zeros((16,), jnp.float32)
    for j in range(_BW // 16):
        r, c = j // 8, (j % 8) * 16
        noise = aidx_v[r, pl.ds(c, 16)] < 0
        gx_v[r, pl.ds(c, 16)] = jnp.where(noise, zero, gx_v[r, pl.ds(c, 16)])
        gy_v[r, pl.ds(c, 16)] = jnp.where(noise, zero, gy_v[r, pl.ds(c, 16)])
        gz_v[r, pl.ds(c, 16)] = jnp.where(noise, zero, gz_v[r, pl.ds(c, 16)])
    pltpu.sync_copy(gx_v, ox_hbm.at[wid])
    pltpu.sync_copy(gy_v, oy_hbm.at[wid])
    pltpu.sync_copy(gz_v, oz_hbm.at[wid])


@functools.partial(
    pl.kernel,
    out_type=(
        jax.ShapeDtypeStruct((_NW, _NCH, 128), jnp.float32),
        jax.ShapeDtypeStruct((_NW, _NCH, 128), jnp.float32),
        jax.ShapeDtypeStruct((_NW, _NCH, 128), jnp.float32),
    ),
    mesh=plsc.VectorSubcoreMesh(core_axis_name="c", subcore_axis_name="s"),
    scratch_types=[
        pltpu.VMEM((_NCH, 128), jnp.int32),
        pltpu.VMEM((_NCH, 128), jnp.int32),
        pltpu.VMEM((_NCH, 128), jnp.float32),
        pltpu.VMEM((_NCH, 128), jnp.float32),
        pltpu.VMEM((_NCH, 128), jnp.float32),
        pltpu.SemaphoreType.DMA,
    ],
)
def _sc_gather(aidx, cx, cy, cz, ox, oy, oz, aidx_v, idx_v, gx, gy, gz, sem):
    _sc_gather_body(aidx, cx, cy, cz, ox, oy, oz,
                    aidx_v, idx_v, gx, gy, gz, sem)


def _thr2(radius):
    r2 = radius * radius
    bits = r2.view(jnp.int32)
    best = jnp.full_like(r2, -1.0)
    for j in range(-8, 9):
        c = jnp.maximum(bits + j, 0).view(jnp.float32)
        keep = jnp.sqrt(c) <= radius
        best = jnp.maximum(best, jnp.where(keep, c, -1.0))
    return best


@jax.jit
def kernel(pred_ccoords, pred_beta, pred_dist):
    pad = _P - _N
    coords = jnp.pad(pred_ccoords, ((0, pad), (0, 0)), constant_values=1e9)
    cxf = coords[:, 0]
    cyf = coords[:, 1]
    czf = coords[:, 2]
    cx = cxf.reshape(_ROWS, _COLS)
    cy = cyf.reshape(_ROWS, _COLS)
    cz = czf.reshape(_ROWS, _COLS)
    beta = jnp.pad(pred_beta.reshape(-1), (0, pad), constant_values=-1.0)
    beta = beta.reshape(_ROWS, _COLS)
    radius = pred_dist.reshape(-1) * _DISTANCE_THRESHOLD
    t2 = jnp.pad(_thr2(radius), (0, pad), constant_values=-1.0)
    t2 = t2.reshape(_ROWS, _COLS)

    shape = (_ROWS, _COLS)
    out_shapes = (
        jax.ShapeDtypeStruct(shape, jnp.int32),
        jax.ShapeDtypeStruct(shape, jnp.int32),
    )
    assign, alpha = pl.pallas_call(
        _greedy_body,
        out_shape=out_shapes,
        scratch_shapes=[pltpu.VMEM(shape, jnp.float32)],
    )(cx, cy, cz, t2, beta)

    aidx = alpha.reshape(_NW, _NCH, 128)
    ox, oy, oz = _sc_gather(aidx, cxf, cyf, czf)

    assign = assign.reshape(-1)[:_N]
    alpha = alpha.reshape(-1)[:_N]
    cond = jnp.stack([ox.reshape(-1)[:_N], oy.reshape(-1)[:_N],
                      oz.reshape(-1)[:_N]], axis=-1)
    return assign, alpha, cond


# speedup vs baseline: 1.1681x; 1.1681x over previous
import functools

import jax
import jax.numpy as jnp
from jax import lax
from jax.experimental import pallas as pl
from jax.experimental.pallas import tpu as pltpu
from jax.experimental.pallas import tpu_sc as plsc

_BETA_THRESHOLD = 0.3
_DISTANCE_THRESHOLD = 0.5
_N = 20000
_ROWS = 160
_COLS = 128
_P = _ROWS * _COLS

_NW = 32
_BW = _P // _NW
_NCH = _BW // 128


def _greedy_body(cx_ref, cy_ref, cz_ref, t2_ref, b_ref,
                 assign_ref, alpha_ref, mb_ref):
    shape = (_ROWS, _COLS)
    neg1 = jnp.full(shape, -1.0, jnp.float32)
    assign_ref[...] = neg1
    alpha_ref[...] = neg1
    mb_ref[...] = b_ref[...]

    row = lax.broadcasted_iota(jnp.int32, shape, 0)
    col = lax.broadcasted_iota(jnp.int32, shape, 1)
    idxf = (row * _COLS + col).astype(jnp.float32)

    cx = cx_ref[...]
    cy = cy_ref[...]
    cz = cz_ref[...]
    t2 = t2_ref[...]

    def cond_fn(carry):
        _, m = carry
        return m > _BETA_THRESHOLD

    def body_fn(carry):
        k, m = carry
        mb = mb_ref[...]
        a = jnp.min(jnp.where(mb >= m, idxf, jnp.float32(_P)))
        sel = idxf == a
        ninf = jnp.float32(-jnp.inf)
        cxa = jnp.max(jnp.where(sel, cx, ninf))
        cya = jnp.max(jnp.where(sel, cy, ninf))
        cza = jnp.max(jnp.where(sel, cz, ninf))
        t2a = jnp.max(jnp.where(sel, t2, ninf))
        dx = cx - cxa
        dy = cy - cya
        dz = cz - cza
        d2 = (dx * dx + dy * dy) + dz * dz
        within = (d2 <= t2a) & (mb >= 0.0)
        assign_ref[...] = jnp.where(within, k, assign_ref[...])
        alpha_ref[...] = jnp.where(within, a, alpha_ref[...])
        mb2 = jnp.where(within, jnp.float32(-2.0), mb)
        mb_ref[...] = mb2
        return k + jnp.float32(1.0), jnp.max(mb2)

    m0 = jnp.max(b_ref[...])
    lax.while_loop(cond_fn, body_fn, (jnp.float32(0.0), m0))


def _sc_gather_body(aidx_hbm, cx_hbm, cy_hbm, cz_hbm,
                    ox_hbm, oy_hbm, oz_hbm,
                    aidx_v, idx_v, gx_v, gy_v, gz_v, sem):
    wid = lax.axis_index("s") * 2 + lax.axis_index("c")
    pltpu.sync_copy(aidx_hbm.at[wid], aidx_v)
    for j in range(_BW // 16):
        r, c = j // 8, (j % 8) * 16
        a = aidx_v[r, pl.ds(c, 16)]
        idx_v[r, pl.ds(c, 16)] = jnp.maximum(a, 0)
    for r in range(_NCH):
        pltpu.async_copy(cx_hbm.at[idx_v.at[r]], gx_v.at[r], sem).wait()
        pltpu.async_copy(cy_hbm.at[idx_v.at[r]], gy_v.at[r], sem).wait()
        pltpu.async_copy(cz_hbm.at[idx_v.at[r]], gz_v.at[r], sem).wait()
    zero = jnp.zeros((16,), jnp.float32)
    for j in range(_BW // 16):
        r, c = j // 8, (j % 8) * 16
        noise = aidx_v[r, pl.ds(c, 16)] < 0
        gx_v[r, pl.ds(c, 16)] = jnp.where(noise, zero, gx_v[r, pl.ds(c, 16)])
        gy_v[r, pl.ds(c, 16)] = jnp.where(noise, zero, gy_v[r, pl.ds(c, 16)])
        gz_v[r, pl.ds(c, 16)] = jnp.where(noise, zero, gz_v[r, pl.ds(c, 16)])
    pltpu.sync_copy(gx_v, ox_hbm.at[wid])
    pltpu.sync_copy(gy_v, oy_hbm.at[wid])
    pltpu.sync_copy(gz_v, oz_hbm.at[wid])


@functools.cache
def _build_sc_gather():
    @functools.partial(
        pl.kernel,
        out_type=(
            jax.ShapeDtypeStruct((_NW, _NCH, 128), jnp.float32),
            jax.ShapeDtypeStruct((_NW, _NCH, 128), jnp.float32),
            jax.ShapeDtypeStruct((_NW, _NCH, 128), jnp.float32),
        ),
        mesh=plsc.VectorSubcoreMesh(core_axis_name="c", subcore_axis_name="s"),
        scratch_types=[
            pltpu.VMEM((_NCH, 128), jnp.int32),
            pltpu.VMEM((_NCH, 128), jnp.int32),
            pltpu.VMEM((_NCH, 128), jnp.float32),
            pltpu.VMEM((_NCH, 128), jnp.float32),
            pltpu.VMEM((_NCH, 128), jnp.float32),
            pltpu.SemaphoreType.DMA,
        ],
    )
    def sc_gather(aidx, cx, cy, cz, ox, oy, oz,
                  aidx_v, idx_v, gx, gy, gz, sem):
        _sc_gather_body(aidx, cx, cy, cz, ox, oy, oz,
                        aidx_v, idx_v, gx, gy, gz, sem)
    return sc_gather


def _sc_gather(aidx, cx, cy, cz):
    return _build_sc_gather()(aidx, cx, cy, cz)


def _thr2(radius):
    r2 = radius * radius
    bits = r2.view(jnp.int32)
    best = jnp.full_like(r2, -1.0)
    for j in range(-8, 9):
        c = jnp.maximum(bits + j, 0).view(jnp.float32)
        keep = jnp.sqrt(c) <= radius
        best = jnp.maximum(best, jnp.where(keep, c, -1.0))
    return best


@jax.jit
def kernel(pred_ccoords, pred_beta, pred_dist):
    pad = _P - _N

    def to2d(flat):
        return flat.reshape(_ROWS, _COLS)

    coords = jnp.pad(pred_ccoords, ((0, pad), (0, 0)), constant_values=1e9)
    cxf = coords[:, 0]
    cyf = coords[:, 1]
    czf = coords[:, 2]
    cx = to2d(cxf)
    cy = to2d(cyf)
    cz = to2d(czf)
    beta = jnp.pad(pred_beta.reshape(-1), (0, pad), constant_values=-1.0)
    beta = to2d(beta)
    radius = pred_dist.reshape(-1) * _DISTANCE_THRESHOLD
    t2 = to2d(jnp.pad(_thr2(radius), (0, pad), constant_values=-1.0))

    shape = (_ROWS, _COLS)
    out_shapes = (
        jax.ShapeDtypeStruct(shape, jnp.float32),
        jax.ShapeDtypeStruct(shape, jnp.float32),
    )
    assign, alpha = pl.pallas_call(
        _greedy_body,
        out_shape=out_shapes,
        scratch_shapes=[pltpu.VMEM(shape, jnp.float32)],
    )(cx, cy, cz, t2, beta)

    assign = assign.reshape(-1).astype(jnp.int32)
    alpha = alpha.reshape(-1).astype(jnp.int32)
    aidx = alpha.reshape(_NW, _NCH, 128)
    ox, oy, oz = _sc_gather(aidx, cxf, cyf, czf)

    assign = assign[:_N]
    alpha = alpha[:_N]
    cond = jnp.stack([ox.reshape(-1)[:_N], oy.reshape(-1)[:_N],
                      oz.reshape(-1)[:_N]], axis=-1)
    return assign, alpha, cond
